# baseline (speedup 1.0000x reference)
import functools

import jax
import jax.numpy as jnp
from jax import lax
from jax.experimental import pallas as pl
from jax.experimental.pallas import tpu as pltpu
from jax.experimental.pallas import tpu_sc as plsc

_B, _C, _T = 16384, 14, 200


def kernel(chromatogram_batch, perm):
    x = chromatogram_batch
    p = perm.astype(jnp.int32)
    cmap = jnp.concatenate([
        p,
        jnp.array([6], jnp.int32),
        p + 7,
        jnp.array([13], jnp.int32),
    ])
    ms = [cmap[c] for c in range(_C)]

    info = plsc.get_sparse_core_info()
    nc = info.num_cores
    half = _B // nc
    mesh = plsc.ScalarSubcoreMesh(axis_name="c", num_cores=nc)

    @functools.partial(
        pl.kernel,
        mesh=mesh,
        out_type=jax.ShapeDtypeStruct((_B, _C, _T), jnp.float32),
        scratch_types=[pltpu.SemaphoreType.DMA],
    )
    def k(x_hbm, out_hbm, sem):
        cid = lax.axis_index("c")
        b0 = cid * half
        copies = []
        for c in range(_C):
            copies.append(
                pltpu.async_copy(
                    x_hbm.at[pl.ds(b0, half), pl.ds(ms[c], 1)],
                    out_hbm.at[pl.ds(b0, half), pl.ds(c, 1)],
                    sem,
                )
            )
        for cp in copies:
            cp.wait()

    return k(x)


# speedup vs baseline: 11.1976x; 11.1976x over previous
import functools

import jax
import jax.numpy as jnp
from jax import lax
from jax.experimental import pallas as pl
from jax.experimental.pallas import tpu as pltpu
from jax.experimental.pallas import tpu_sc as plsc

_B, _C, _T = 16384, 14, 200
_NB = 128


def kernel(chromatogram_batch, perm):
    x = chromatogram_batch
    p = perm.astype(jnp.int32)
    cmap = jnp.concatenate([
        p,
        jnp.array([6], jnp.int32),
        p + 7,
        jnp.array([13], jnp.int32),
        jnp.array([0, 0], jnp.int32),
    ])

    info = plsc.get_sparse_core_info()
    nc, ns = info.num_cores, info.num_subcores
    nw = nc * ns
    bw = _B // nw
    nchunks = bw // _NB
    mesh = plsc.VectorSubcoreMesh(core_axis_name="c", subcore_axis_name="s")

    @functools.partial(
        pl.kernel,
        mesh=mesh,
        out_type=jax.ShapeDtypeStruct((_B, _C, _T), jnp.float32),
        compiler_params=pltpu.CompilerParams(needs_layout_passes=False),
        scratch_types=[
            pltpu.VMEM((16,), jnp.int32),
            pltpu.VMEM((_NB, 1, _T), jnp.float32),
            pltpu.SemaphoreType.DMA,
        ],
    )
    def k(x_hbm, cmap_hbm, out_hbm, cmap_v, buf_v, sem):
        wid = lax.axis_index("s") * nc + lax.axis_index("c")
        b0 = wid * bw
        pltpu.sync_copy(cmap_hbm, cmap_v)
        cmapv = cmap_v[...]
        lane = lax.broadcasted_iota(jnp.int32, (16,), 0)
        for c in range(_C):
            src = jnp.sum(jnp.where(lane == c, cmapv, 0), axis=0)
            for j in range(nchunks):
                base = b0 + j * _NB
                pltpu.async_copy(
                    x_hbm.at[pl.ds(base, _NB), pl.ds(src, 1)], buf_v, sem
                ).wait()
                pltpu.sync_copy(buf_v, out_hbm.at[pl.ds(base, _NB), pl.ds(c, 1)])

    return k(x, cmap)


# speedup vs baseline: 11.8236x; 1.0559x over previous
import functools

import jax
import jax.numpy as jnp
from jax import lax
from jax.experimental import pallas as pl
from jax.experimental.pallas import tpu as pltpu
from jax.experimental.pallas import tpu_sc as plsc

_B, _C, _T = 16384, 14, 200
_NB = 64
_K = 6
_D = 3


def kernel(chromatogram_batch, perm):
    x = chromatogram_batch
    p = perm.astype(jnp.int32)
    cmap = jnp.concatenate([
        p,
        jnp.array([6], jnp.int32),
        p + 7,
        jnp.array([13], jnp.int32),
        jnp.array([0, 0], jnp.int32),
    ])

    info = plsc.get_sparse_core_info()
    nc, ns = info.num_cores, info.num_subcores
    nw = nc * ns
    bw = _B // nw
    nchunks = bw // _NB
    mesh = plsc.VectorSubcoreMesh(core_axis_name="c", subcore_axis_name="s")

    @functools.partial(
        pl.kernel,
        mesh=mesh,
        out_type=jax.ShapeDtypeStruct((_B, _C, _T), jnp.float32),
        compiler_params=pltpu.CompilerParams(needs_layout_passes=False),
        scratch_types=[
            pltpu.VMEM((16,), jnp.int32),
            *[pltpu.VMEM((_NB, 1, _T), jnp.float32) for _ in range(_K)],
            *[pltpu.SemaphoreType.DMA for _ in range(2 * _K)],
        ],
    )
    def k(x_hbm, cmap_hbm, out_hbm, cmap_v, *rest):
        bufs = rest[:_K]
        gsem = rest[_K:2 * _K]
        wsem = rest[2 * _K:3 * _K]
        wid = lax.axis_index("s") * nc + lax.axis_index("c")
        b0 = wid * bw
        pltpu.sync_copy(cmap_hbm, cmap_v)
        cmapv = cmap_v[...]
        lane = lax.broadcasted_iota(jnp.int32, (16,), 0)
        srcs = [
            jnp.sum(jnp.where(lane == c, cmapv, 0), axis=0) for c in range(_C)
        ]
        steps = [(c, j) for c in range(_C) for j in range(nchunks)]
        n = len(steps)

        def fire_gather(i):
            c, j = steps[i]
            return pltpu.async_copy(
                x_hbm.at[pl.ds(b0 + j * _NB, _NB), pl.ds(srcs[c], 1)],
                bufs[i % _K],
                gsem[i % _K],
            )

        def fire_write(i):
            c, j = steps[i]
            return pltpu.async_copy(
                bufs[i % _K],
                out_hbm.at[pl.ds(b0 + j * _NB, _NB), pl.ds(c, 1)],
                wsem[i % _K],
            )

        gc = [None] * n
        wc = [None] * n
        for i in range(_D):
            gc[i] = fire_gather(i)
        for i in range(n):
            f = i + _D
            if f < n:
                if f >= _K:
                    wc[f - _K].wait()
                gc[f] = fire_gather(f)
            gc[i].wait()
            wc[i] = fire_write(i)
        for i in range(max(0, n - _K), n):
            wc[i].wait()

    return k(x, cmap)
